# Initial kernel scaffold; baseline (speedup 1.0000x reference)
import jax
import jax.numpy as jnp
from jax.experimental import pallas as pl


def kernel(input_ids, position_ids, subword_table, m1_table, m2_table):
    raise NotImplementedError("write your pallas kernel here")


# speedup vs baseline: 2.9924x; 2.9924x over previous
import functools

import jax
import jax.numpy as jnp
from jax import lax
from jax.experimental import pallas as pl
from jax.experimental.pallas import tpu as pltpu
from jax.experimental.pallas import tpu_sc as plsc

DIM = 64
LPAD = 208
CHUNK = 1024
RPG = 128
NC = 2
NS = 16
NW = NC * NS


def _embed_body(ids_hbm, pos_hbm, tab_hbm, m1_hbm, m2_hbm, out_hbm,
                m1_v, m2_v, pt_v, idx_v, pos_v, rows_v, gsem):
    wid = lax.axis_index("s") * NC + lax.axis_index("c")
    n = out_hbm.shape[0]
    n_per_w = n // NW
    n_chunks = n_per_w // CHUNK
    base = wid * n_per_w

    pltpu.sync_copy(m1_hbm, m1_v)
    pltpu.sync_copy(m2_hbm, m2_v)
    col = [jnp.full((16,), c, jnp.int32) for c in range(DIM)]

    def build_grp(g, carry):
        p = lax.iota(jnp.int32, 16) + g * 16
        i47 = lax.rem(p, 47)
        i11 = lax.rem(p, 11)
        pbase = p * DIM
        for c in range(DIM):
            v = plsc.load_gather(m1_v, [i47, col[c]]) + plsc.load_gather(m2_v, [i11, col[c]])
            plsc.store_scatter(pt_v, [pbase + c], v)
        return carry

    lax.fori_loop(0, LPAD // 16, build_grp, 0)


    def chunk_body(k, carry):
        off = pl.multiple_of(base + k * CHUNK, CHUNK)
        row_off = pl.multiple_of(base // RPG + k * (CHUNK // RPG), CHUNK // RPG)
        pltpu.sync_copy(ids_hbm.at[pl.ds(row_off, CHUNK // RPG)], idx_v)
        pltpu.sync_copy(pos_hbm.at[pl.ds(off, CHUNK)], pos_v)
        cps = [
            pltpu.async_copy(tab_hbm.at[idx_v.at[r]],
                             rows_v.at[pl.ds(r * RPG, RPG)], gsem)
            for r in range(CHUNK // RPG)
        ]
        for cp in cps:
            cp.wait()

        def grp(g, c2):
            p = pos_v[pl.ds(g * 16, 16)]
            pidx = p * DIM
            ridx = lax.iota(jnp.int32, 16) + g * 16
            for c in range(DIM):
                v = plsc.load_gather(pt_v, [pidx + c])
                plsc.addupdate_scatter(rows_v, [ridx, col[c]], v)
            return c2

        lax.fori_loop(0, CHUNK // 16, grp, 0)
        pltpu.sync_copy(rows_v, out_hbm.at[pl.ds(off, CHUNK)])
        return carry

    lax.fori_loop(0, n_chunks, chunk_body, 0)


@jax.jit
def _sc_embed(ids2d, pos_flat, subword_table, m1_table, m2_table):
    n = pos_flat.shape[0]
    mesh = plsc.VectorSubcoreMesh(core_axis_name="c", subcore_axis_name="s")
    f = pl.kernel(
        _embed_body,
        out_type=jax.ShapeDtypeStruct((n, DIM), jnp.float32),
        mesh=mesh,
        compiler_params=pltpu.CompilerParams(
            needs_layout_passes=False, use_tc_tiling_on_sc=False),
        scratch_types=[
            pltpu.VMEM((47, DIM), jnp.float32),
            pltpu.VMEM((11, DIM), jnp.float32),
            pltpu.VMEM((LPAD * DIM,), jnp.float32),
            pltpu.VMEM((CHUNK // RPG, RPG), jnp.int32),
            pltpu.VMEM((CHUNK,), jnp.int32),
            pltpu.VMEM((CHUNK, DIM), jnp.float32),
            pltpu.SemaphoreType.DMA,
        ],
    )
    return f(ids2d, pos_flat, subword_table, m1_table, m2_table)


def kernel(input_ids, position_ids, subword_table, m1_table, m2_table):
    b, l = input_ids.shape
    n = b * l
    ids2d = input_ids.reshape(n // RPG, RPG)
    pos_flat = position_ids.reshape(n)
    out = _sc_embed(ids2d, pos_flat, subword_table, m1_table, m2_table)
    return out.reshape(b, l, DIM)


# speedup vs baseline: 3.7021x; 1.2372x over previous
import functools

import jax
import jax.numpy as jnp
from jax import lax
from jax.experimental import pallas as pl
from jax.experimental.pallas import tpu as pltpu
from jax.experimental.pallas import tpu_sc as plsc

DIM = 64
LPAD = 208
CHUNK = 1024
RPG = 128
NC = 2
NS = 16
NW = NC * NS


def _embed_body(ids_hbm, pos_hbm, tab_hbm, m1_hbm, m2_hbm, out_hbm,
                m1_v, m2_v, pt_v, idx_v, pos_v, rows_v, gsem, wsem):
    wid = lax.axis_index("s") * NC + lax.axis_index("c")
    n = out_hbm.shape[0]
    n_per_w = n // NW
    n_chunks = n_per_w // CHUNK
    base = wid * n_per_w

    pltpu.sync_copy(m1_hbm, m1_v)
    pltpu.sync_copy(m2_hbm, m2_v)
    col = [jnp.full((16,), c, jnp.int32) for c in range(DIM)]

    def build_grp(g, carry):
        p = lax.iota(jnp.int32, 16) + g * 16
        i47 = lax.rem(p, 47)
        i11 = lax.rem(p, 11)
        pbase = p * DIM
        for c in range(DIM):
            v = plsc.load_gather(m1_v, [i47, col[c]]) + plsc.load_gather(m2_v, [i11, col[c]])
            plsc.store_scatter(pt_v, [pbase + c], v)
        return carry

    lax.fori_loop(0, LPAD // 16, build_grp, 0)


    half = CHUNK // 2

    def chunk_body(k, carry):
        off = pl.multiple_of(base + k * CHUNK, CHUNK)
        row_off = pl.multiple_of(base // RPG + k * (CHUNK // RPG), CHUNK // RPG)
        pltpu.sync_copy(ids_hbm.at[pl.ds(row_off, CHUNK // RPG)], idx_v)
        pltpu.sync_copy(pos_hbm.at[pl.ds(off, CHUNK)], pos_v)
        gcps = [
            [pltpu.async_copy(tab_hbm.at[idx_v.at[h * (half // RPG) + r]],
                              rows_v.at[h].at[pl.ds(r * RPG, RPG)],
                              gsem[h])
             for r in range(half // RPG)]
            for h in range(2)
        ]
        wcps = []
        for h in range(2):
            for cp in gcps[h]:
                cp.wait()
            half_ref = rows_v.at[h]

            def grp(g, c2):
                p = pos_v[pl.ds(h * half + g * 16, 16)]
                pidx = p * DIM
                ridx = lax.iota(jnp.int32, 16) + g * 16
                for c0 in range(0, DIM, 8):
                    vs = [plsc.load_gather(pt_v, [pidx + (c0 + i)])
                          for i in range(8)]
                    for i in range(8):
                        plsc.addupdate_scatter(half_ref, [ridx, col[c0 + i]], vs[i])
                return c2

            lax.fori_loop(0, half // 16, grp, 0)
            wcps.append(pltpu.async_copy(
                half_ref, out_hbm.at[pl.ds(off + h * half, half)], wsem))
        for cp in wcps:
            cp.wait()
        return carry

    lax.fori_loop(0, n_chunks, chunk_body, 0)


@jax.jit
def _sc_embed(ids2d, pos_flat, subword_table, m1_table, m2_table):
    n = pos_flat.shape[0]
    mesh = plsc.VectorSubcoreMesh(core_axis_name="c", subcore_axis_name="s")
    f = pl.kernel(
        _embed_body,
        out_type=jax.ShapeDtypeStruct((n, DIM), jnp.float32),
        mesh=mesh,
        compiler_params=pltpu.CompilerParams(
            needs_layout_passes=False, use_tc_tiling_on_sc=False),
        scratch_types=[
            pltpu.VMEM((47, DIM), jnp.float32),
            pltpu.VMEM((11, DIM), jnp.float32),
            pltpu.VMEM((LPAD * DIM,), jnp.float32),
            pltpu.VMEM((CHUNK // RPG, RPG), jnp.int32),
            pltpu.VMEM((CHUNK,), jnp.int32),
            pltpu.VMEM((2, CHUNK // 2, DIM), jnp.float32),
            [pltpu.SemaphoreType.DMA, pltpu.SemaphoreType.DMA],
            pltpu.SemaphoreType.DMA,
        ],
    )
    return f(ids2d, pos_flat, subword_table, m1_table, m2_table)


def kernel(input_ids, position_ids, subword_table, m1_table, m2_table):
    b, l = input_ids.shape
    n = b * l
    ids2d = input_ids.reshape(n // RPG, RPG)
    pos_flat = position_ids.reshape(n)
    out = _sc_embed(ids2d, pos_flat, subword_table, m1_table, m2_table)
    return out.reshape(b, l, DIM)


# speedup vs baseline: 9.6403x; 2.6040x over previous
import functools

import jax
import jax.numpy as jnp
from jax import lax
from jax.experimental import pallas as pl
from jax.experimental.pallas import tpu as pltpu
from jax.experimental.pallas import tpu_sc as plsc

DIM = 64
PTS = 65
LPAD = 208
CHUNK = 1024
RPG = 128
NC = 2
NS = 16
NW = NC * NS


def _embed_body(ids_hbm, pos_hbm, tab_hbm, m1_hbm, m2_hbm, out_hbm,
                m1_v, m2_v, pt_v, idx_v, pos_v, rows_v, gsem, wsem):
    wid = lax.axis_index("s") * NC + lax.axis_index("c")
    n = out_hbm.shape[0]
    n_per_w = n // NW
    n_chunks = n_per_w // CHUNK
    base = wid * n_per_w

    pltpu.sync_copy(m1_hbm, m1_v)
    pltpu.sync_copy(m2_hbm, m2_v)

    def build_grp(g, carry):
        p = lax.iota(jnp.int32, 16) + g * 16
        coff = lax.iota(jnp.int32, 16)
        i47 = lax.rem(p, 47)
        i11 = lax.rem(p, 11)
        pbase = p * PTS
        for c in range(DIM):
            colv = (coff + c) & 63
            v = plsc.load_gather(m1_v, [i47, colv]) + plsc.load_gather(m2_v, [i11, colv])
            plsc.store_scatter(pt_v, [pbase + colv], v)
        return carry

    lax.fori_loop(0, LPAD // 16, build_grp, 0)


    half = CHUNK // 2

    def chunk_body(k, carry):
        off = pl.multiple_of(base + k * CHUNK, CHUNK)
        row_off = pl.multiple_of(base // RPG + k * (CHUNK // RPG), CHUNK // RPG)
        pltpu.sync_copy(ids_hbm.at[pl.ds(row_off, CHUNK // RPG)], idx_v)
        pltpu.sync_copy(pos_hbm.at[pl.ds(off, CHUNK)], pos_v)
        gcps = [
            [pltpu.async_copy(tab_hbm.at[idx_v.at[h * (half // RPG) + r]],
                              rows_v.at[h].at[pl.ds(r * RPG, RPG)],
                              gsem[h])
             for r in range(half // RPG)]
            for h in range(2)
        ]
        wcps = []
        for h in range(2):
            for cp in gcps[h]:
                cp.wait()
            half_ref = rows_v.at[h]

            def grp(g, c2):
                p = pos_v[pl.ds(h * half + g * 16, 16)]
                coff = lax.iota(jnp.int32, 16)
                pidx = p * PTS
                ridx = coff + g * 16
                for c0 in range(0, DIM, 8):
                    cols = [(coff + (c0 + i)) & 63 for i in range(8)]
                    vs = [plsc.load_gather(pt_v, [pidx + cols[i]])
                          for i in range(8)]
                    for i in range(8):
                        plsc.addupdate_scatter(half_ref, [ridx, cols[i]], vs[i])
                return c2

            lax.fori_loop(0, half // 16, grp, 0)
            wcps.append(pltpu.async_copy(
                half_ref, out_hbm.at[pl.ds(off + h * half, half)], wsem))
        for cp in wcps:
            cp.wait()
        return carry

    lax.fori_loop(0, n_chunks, chunk_body, 0)


@jax.jit
def _sc_embed(ids2d, pos_flat, subword_table, m1_table, m2_table):
    n = pos_flat.shape[0]
    mesh = plsc.VectorSubcoreMesh(core_axis_name="c", subcore_axis_name="s")
    f = pl.kernel(
        _embed_body,
        out_type=jax.ShapeDtypeStruct((n, DIM), jnp.float32),
        mesh=mesh,
        compiler_params=pltpu.CompilerParams(
            needs_layout_passes=False, use_tc_tiling_on_sc=False),
        scratch_types=[
            pltpu.VMEM((47, DIM), jnp.float32),
            pltpu.VMEM((11, DIM), jnp.float32),
            pltpu.VMEM((LPAD * PTS,), jnp.float32),
            pltpu.VMEM((CHUNK // RPG, RPG), jnp.int32),
            pltpu.VMEM((CHUNK,), jnp.int32),
            pltpu.VMEM((2, CHUNK // 2, DIM), jnp.float32),
            [pltpu.SemaphoreType.DMA, pltpu.SemaphoreType.DMA],
            pltpu.SemaphoreType.DMA,
        ],
    )
    return f(ids2d, pos_flat, subword_table, m1_table, m2_table)


def kernel(input_ids, position_ids, subword_table, m1_table, m2_table):
    b, l = input_ids.shape
    n = b * l
    ids2d = input_ids.reshape(n // RPG, RPG)
    pos_flat = position_ids.reshape(n)
    out = _sc_embed(ids2d, pos_flat, subword_table, m1_table, m2_table)
    return out.reshape(b, l, DIM)


# speedup vs baseline: 9.7814x; 1.0146x over previous
import functools

import jax
import jax.numpy as jnp
from jax import lax
from jax.experimental import pallas as pl
from jax.experimental.pallas import tpu as pltpu
from jax.experimental.pallas import tpu_sc as plsc

DIM = 64
PTS = 65
LPAD = 208
CHUNK = 1024
RPG = 128
NC = 2
NS = 16
NW = NC * NS


def _embed_body(ids_hbm, pos_hbm, tab_hbm, m1_hbm, m2_hbm, out_hbm,
                m1_v, m2_v, pt_v, idx_v, pos_v, rows_v, gsem, wsem):
    wid = lax.axis_index("s") * NC + lax.axis_index("c")
    n = out_hbm.shape[0]
    n_per_w = n // NW
    n_chunks = n_per_w // CHUNK
    base = wid * n_per_w

    pltpu.sync_copy(m1_hbm, m1_v)
    pltpu.sync_copy(m2_hbm, m2_v)

    def build_grp(g, carry):
        p = lax.iota(jnp.int32, 16) + g * 16
        coff = lax.iota(jnp.int32, 16)
        i47 = lax.rem(p, 47)
        i11 = lax.rem(p, 11)
        pbase = p * PTS
        for c in range(DIM):
            colv = (coff + c) & 63
            v = plsc.load_gather(m1_v, [i47, colv]) + plsc.load_gather(m2_v, [i11, colv])
            plsc.store_scatter(pt_v, [pbase + colv], v)
        return carry

    lax.fori_loop(0, LPAD // 16, build_grp, 0)


    half = CHUNK // 2

    def chunk_body(k, carry):
        off = pl.multiple_of(base + k * CHUNK, CHUNK)
        row_off = pl.multiple_of(base // RPG + k * (CHUNK // RPG), CHUNK // RPG)
        pltpu.sync_copy(ids_hbm.at[pl.ds(row_off, CHUNK // RPG)], idx_v)
        gcps = [
            [pltpu.async_copy(tab_hbm.at[idx_v.at[h * (half // RPG) + r]],
                              rows_v.at[h].at[pl.ds(r * RPG, RPG)],
                              gsem[h])
             for r in range(half // RPG)]
            for h in range(2)
        ]
        pltpu.sync_copy(pos_hbm.at[pl.ds(off, CHUNK)], pos_v)
        wcps = []
        for h in range(2):
            for cp in gcps[h]:
                cp.wait()
            half_ref = rows_v.at[h]

            def grp(g, c2):
                p = pos_v[pl.ds(h * half + g * 16, 16)]
                coff = lax.iota(jnp.int32, 16)
                pidx = p * PTS
                ridx = coff + g * 16
                for c0 in range(0, DIM, 8):
                    cols = [(coff + (c0 + i)) & 63 for i in range(8)]
                    vs = [plsc.load_gather(pt_v, [pidx + cols[i]])
                          for i in range(8)]
                    for i in range(8):
                        plsc.addupdate_scatter(half_ref, [ridx, cols[i]], vs[i])
                return c2

            lax.fori_loop(0, half // 16, grp, 0)
            wcps.append(pltpu.async_copy(
                half_ref, out_hbm.at[pl.ds(off + h * half, half)], wsem))
        for cp in wcps:
            cp.wait()
        return carry

    lax.fori_loop(0, n_chunks, chunk_body, 0)


@jax.jit
def _sc_embed(ids2d, pos_flat, subword_table, m1_table, m2_table):
    n = pos_flat.shape[0]
    mesh = plsc.VectorSubcoreMesh(core_axis_name="c", subcore_axis_name="s")
    f = pl.kernel(
        _embed_body,
        out_type=jax.ShapeDtypeStruct((n, DIM), jnp.float32),
        mesh=mesh,
        compiler_params=pltpu.CompilerParams(
            needs_layout_passes=False, use_tc_tiling_on_sc=False),
        scratch_types=[
            pltpu.VMEM((47, DIM), jnp.float32),
            pltpu.VMEM((11, DIM), jnp.float32),
            pltpu.VMEM((LPAD * PTS,), jnp.float32),
            pltpu.VMEM((CHUNK // RPG, RPG), jnp.int32),
            pltpu.VMEM((CHUNK,), jnp.int32),
            pltpu.VMEM((2, CHUNK // 2, DIM), jnp.float32),
            [pltpu.SemaphoreType.DMA, pltpu.SemaphoreType.DMA],
            pltpu.SemaphoreType.DMA,
        ],
    )
    return f(ids2d, pos_flat, subword_table, m1_table, m2_table)


def kernel(input_ids, position_ids, subword_table, m1_table, m2_table):
    b, l = input_ids.shape
    n = b * l
    ids2d = input_ids.reshape(n // RPG, RPG)
    pos_flat = position_ids.reshape(n)
    out = _sc_embed(ids2d, pos_flat, subword_table, m1_table, m2_table)
    return out.reshape(b, l, DIM)
